# Initial kernel scaffold; baseline (speedup 1.0000x reference)
import jax
import jax.numpy as jnp
from jax.experimental import pallas as pl


def kernel(xyz, feats, affine_alpha_p, affine_beta_p):
    raise NotImplementedError("write your pallas kernel here")


# speedup vs baseline: 1.9548x; 1.9548x over previous
import functools

import jax
import jax.numpy as jnp
from jax.experimental import pallas as pl
from jax.experimental.pallas import tpu as pltpu

B, N, S, K, C = 8, 2048, 512, 32, 128


def _fps_body(xyz_ref, fps_ref, nxyz_ref):
    x = xyz_ref[:, 0, :]
    y = xyz_ref[:, 1, :]
    z = xyz_ref[:, 2, :]
    lane_n = jax.lax.broadcasted_iota(jnp.int32, (B, N), 1)
    lane_s = jax.lax.broadcasted_iota(jnp.int32, (B, S), 1)

    def body(i, carry):
        dist, far = carry
        fps_ref[...] = jnp.where(lane_s == i, far, fps_ref[...])
        m = lane_n == far
        cx = jnp.sum(jnp.where(m, x, 0.0), 1, keepdims=True)
        cy = jnp.sum(jnp.where(m, y, 0.0), 1, keepdims=True)
        cz = jnp.sum(jnp.where(m, z, 0.0), 1, keepdims=True)
        nxyz_ref[:, 0, :] = jnp.where(lane_s == i, cx, nxyz_ref[:, 0, :])
        nxyz_ref[:, 1, :] = jnp.where(lane_s == i, cy, nxyz_ref[:, 1, :])
        nxyz_ref[:, 2, :] = jnp.where(lane_s == i, cz, nxyz_ref[:, 2, :])
        d = (x - cx) ** 2 + (y - cy) ** 2 + (z - cz) ** 2
        dist = jnp.minimum(dist, d)
        mx = jnp.max(dist, 1, keepdims=True)
        far2 = jnp.min(jnp.where(dist == mx, lane_n, N), 1, keepdims=True)
        return dist, far2

    dist0 = jnp.full((B, N), 1e10, jnp.float32)
    far0 = jnp.zeros((B, 1), jnp.int32)
    jax.lax.fori_loop(0, S, body, (dist0, far0))


_fps_pallas = pl.pallas_call(
    _fps_body,
    out_shape=(
        jax.ShapeDtypeStruct((B, S), jnp.int32),
        jax.ShapeDtypeStruct((B, 3, S), jnp.float32),
    ),
)


def kernel(xyz, feats, affine_alpha_p, affine_beta_p):
    fps_idx, new_xyz = _fps_pallas(xyz)
    xyz_t = jnp.swapaxes(xyz, 1, 2)
    feats_t = jnp.swapaxes(feats, 1, 2)
    new_xyz_t = jnp.swapaxes(new_xyz, 1, 2)

    new_feats = jnp.take_along_axis(feats_t, fps_idx[:, :, None], axis=1)

    sqr = -2.0 * jnp.matmul(new_xyz_t, jnp.swapaxes(xyz_t, 1, 2))
    sqr = sqr + jnp.sum(new_xyz_t ** 2, -1)[:, :, None]
    sqr = sqr + jnp.sum(xyz_t ** 2, -1)[:, None, :]
    _, idx = jax.lax.top_k(-sqr, K)

    grouped = jnp.take_along_axis(
        feats_t, idx.reshape(B, -1)[:, :, None], axis=1
    ).reshape(B, S, K, C)
    mean = new_feats[:, :, None, :]
    diff = grouped - mean
    std = jnp.std(diff.reshape(B, -1), axis=-1, ddof=1, keepdims=True)[:, :, None, None]
    grouped = diff / (std + 1e-05)
    grouped = affine_alpha_p * grouped + affine_beta_p
    center = jnp.broadcast_to(new_feats.reshape(B, S, 1, C), grouped.shape)
    point_feats = jnp.concatenate([grouped, grouped - center], axis=-1)
    new_feats_o = jnp.transpose(new_feats, (0, 2, 1))
    point_feats = jnp.transpose(point_feats, (0, 3, 1, 2))
    return (new_xyz, new_feats_o, point_feats)


# speedup vs baseline: 2.7430x; 1.4032x over previous
import functools

import jax
import jax.numpy as jnp
from jax.experimental import pallas as pl
from jax.experimental.pallas import tpu as pltpu

B, N, S, K, C = 8, 2048, 512, 32, 128


def _fps_body(xyz_ref, fps_ref, nxyz_ref):
    x = xyz_ref[:, 0, :]
    y = xyz_ref[:, 1, :]
    z = xyz_ref[:, 2, :]
    lane_n = jax.lax.broadcasted_iota(jnp.int32, (B, N), 1)
    lane_s = jax.lax.broadcasted_iota(jnp.int32, (B, S), 1)

    def body(i, carry):
        dist, far = carry
        fps_ref[...] = jnp.where(lane_s == i, far, fps_ref[...])
        m = lane_n == far
        cx = jnp.sum(jnp.where(m, x, 0.0), 1, keepdims=True)
        cy = jnp.sum(jnp.where(m, y, 0.0), 1, keepdims=True)
        cz = jnp.sum(jnp.where(m, z, 0.0), 1, keepdims=True)
        nxyz_ref[:, 0, :] = jnp.where(lane_s == i, cx, nxyz_ref[:, 0, :])
        nxyz_ref[:, 1, :] = jnp.where(lane_s == i, cy, nxyz_ref[:, 1, :])
        nxyz_ref[:, 2, :] = jnp.where(lane_s == i, cz, nxyz_ref[:, 2, :])
        d = (x - cx) ** 2 + (y - cy) ** 2 + (z - cz) ** 2
        dist = jnp.minimum(dist, d)
        mx = jnp.max(dist, 1, keepdims=True)
        far2 = jnp.min(jnp.where(dist == mx, lane_n, N), 1, keepdims=True)
        return dist, far2

    dist0 = jnp.full((B, N), 1e10, jnp.float32)
    far0 = jnp.zeros((B, 1), jnp.int32)
    jax.lax.fori_loop(0, S, body, (dist0, far0))


_fps_pallas = pl.pallas_call(
    _fps_body,
    out_shape=(
        jax.ShapeDtypeStruct((B, S), jnp.int32),
        jax.ShapeDtypeStruct((B, 3, S), jnp.float32),
    ),
)


def _knn_body(nx_ref, xyz_ref, idx_ref):
    a = nx_ref[0]
    bx = xyz_ref[0]
    d = jax.lax.dot_general(a, bx, (((0,), (0,)), ((), ())),
                            preferred_element_type=jnp.float32)
    d = -2.0 * d
    d = d + jnp.sum(a * a, 0)[:, None]
    d = d + jnp.sum(bx * bx, 0)[None, :]
    lane = jax.lax.broadcasted_iota(jnp.int32, (S, N), 1)
    cols = []
    for _ in range(K):
        m = jnp.min(d, 1, keepdims=True)
        am = jnp.min(jnp.where(d == m, lane, N), 1, keepdims=True)
        d = jnp.where(lane == am, jnp.inf, d)
        cols.append(am)
    idx_ref[0] = jnp.concatenate(cols, axis=1)


_knn_pallas = pl.pallas_call(
    _knn_body,
    grid=(B,),
    in_specs=[
        pl.BlockSpec((1, 3, S), lambda b: (b, 0, 0)),
        pl.BlockSpec((1, 3, N), lambda b: (b, 0, 0)),
    ],
    out_specs=pl.BlockSpec((1, S, K), lambda b: (b, 0, 0)),
    out_shape=jax.ShapeDtypeStruct((B, S, K), jnp.int32),
)


def kernel(xyz, feats, affine_alpha_p, affine_beta_p):
    fps_idx, new_xyz = _fps_pallas(xyz)
    feats_t = jnp.swapaxes(feats, 1, 2)

    new_feats = jnp.take_along_axis(feats_t, fps_idx[:, :, None], axis=1)

    idx = _knn_pallas(new_xyz, xyz)

    grouped = jnp.take_along_axis(
        feats_t, idx.reshape(B, -1)[:, :, None], axis=1
    ).reshape(B, S, K, C)
    mean = new_feats[:, :, None, :]
    diff = grouped - mean
    std = jnp.std(diff.reshape(B, -1), axis=-1, ddof=1, keepdims=True)[:, :, None, None]
    grouped = diff / (std + 1e-05)
    grouped = affine_alpha_p * grouped + affine_beta_p
    center = jnp.broadcast_to(new_feats.reshape(B, S, 1, C), grouped.shape)
    point_feats = jnp.concatenate([grouped, grouped - center], axis=-1)
    new_feats_o = jnp.transpose(new_feats, (0, 2, 1))
    point_feats = jnp.transpose(point_feats, (0, 3, 1, 2))
    return (new_xyz, new_feats_o, point_feats)
